# baseline (speedup 1.0000x reference)
import functools

import jax
import jax.numpy as jnp
from jax import lax
from jax.experimental import pallas as pl
from jax.experimental.pallas import tpu as pltpu
from jax.experimental.pallas import tpu_sc as plsc

SEQ, B = 4096, 4
NUM_SEGMENTS = 8
EMB_DIM = 128
OUT_DIM = 1024
N_TOKENS = SEQ * B

NC, NS = 2, 16
NW = NC * NS
TOK_PER_W = N_TOKENS // NW
CH = 64
NCH = TOK_PER_W // CH


def _p_kernel(table_ref, w_ref, p_ref):
    p_ref[...] = lax.dot_general(
        table_ref[...], w_ref[...],
        dimension_numbers=(((1,), (1,)), ((), ())),
        preferred_element_type=jnp.float32,
    )


def _sc_body(p_hbm, seg_hbm, out_hbm, idx_v, rows_v, sem):
    wid = lax.axis_index("s") * NC + lax.axis_index("c")
    base = wid * TOK_PER_W
    pltpu.sync_copy(seg_hbm.at[wid], idx_v)
    for j in range(NCH):
        pltpu.async_copy(p_hbm.at[idx_v.at[j]], rows_v, sem).wait()
        pltpu.sync_copy(rows_v, out_hbm.at[pl.ds(base + j * CH, CH)])


@jax.jit
def kernel(input, align_pos, segment_ids, table, W):
    seg = segment_ids.astype(jnp.int32).reshape(NW, NCH, CH)
    P = pl.pallas_call(
        _p_kernel,
        out_shape=jax.ShapeDtypeStruct((NUM_SEGMENTS, OUT_DIM), jnp.float32),
    )(table, W)

    sc_gather = functools.partial(
        pl.kernel,
        out_type=jax.ShapeDtypeStruct((N_TOKENS, OUT_DIM), jnp.float32),
        mesh=plsc.VectorSubcoreMesh(core_axis_name="c", subcore_axis_name="s"),
        scratch_types=[
            pltpu.VMEM((NCH, CH), jnp.int32),
            pltpu.VMEM((CH, OUT_DIM), jnp.float32),
            pltpu.SemaphoreType.DMA,
        ],
    )(_sc_body)
    out = sc_gather(P, seg)
    return out.reshape(SEQ, B, OUT_DIM)
